# baseline (speedup 1.0000x reference)
import functools

import jax
import jax.numpy as jnp
from jax.experimental import pallas as pl
from jax.experimental.pallas import tpu as pltpu

EPS = 1e-05
INV_D = 30.0
H = W = 16
SKIP = 2
N_KEEP = (H - 2 * SKIP) * (W - 2 * SKIP)


def _moca_kernel(x_ref, emb_ref, codes_ref, bow_ref):
    xv = x_ref[0]
    n = jnp.sqrt(jnp.sum(xv * xv, axis=1, keepdims=True))
    xn = (xv / jnp.maximum(n, EPS)).astype(jnp.bfloat16)
    logits = INV_D * jax.lax.dot_general(
        xn, emb_ref[...],
        dimension_numbers=(((1,), (1,)), ((), ())),
        preferred_element_type=jnp.float32,
    )
    m = jnp.max(logits, axis=1, keepdims=True)
    e = jnp.exp(logits - m)
    s = jnp.sum(e, axis=1, keepdims=True)
    codes = e / s
    codes_ref[0] = codes

    t = jax.lax.broadcasted_iota(jnp.int32, (256, 1), 0)
    r = t // W
    c = t % W
    keep = (r >= SKIP) & (r < H - SKIP) & (c >= SKIP) & (c < W - SKIP)
    mask = keep.astype(jnp.float32)
    bow = jnp.sum(codes * mask, axis=0, keepdims=True) / N_KEEP
    l1 = jnp.sum(jnp.abs(bow))
    bow_ref[0] = bow / jnp.maximum(l1, EPS)


@jax.jit
def kernel(x, embedding):
    B = x.shape[0]
    xs = x[:, 1:, :]
    L = xs.shape[1]
    K = embedding.shape[0]
    embedding = embedding.astype(jnp.bfloat16)
    codes, bow = pl.pallas_call(
        _moca_kernel,
        grid=(B,),
        in_specs=[
            pl.BlockSpec((1, L, xs.shape[2]), lambda b: (b, 0, 0)),
            pl.BlockSpec((K, xs.shape[2]), lambda b: (0, 0)),
        ],
        out_specs=[
            pl.BlockSpec((1, L, K), lambda b: (b, 0, 0)),
            pl.BlockSpec((1, 1, K), lambda b: (b, 0, 0)),
        ],
        out_shape=[
            jax.ShapeDtypeStruct((B, L, K), jnp.float32),
            jax.ShapeDtypeStruct((B, 1, K), jnp.float32),
        ],
    )(xs, embedding)
    return (bow.reshape(B, K), codes)


# speedup vs baseline: 1.2551x; 1.2551x over previous
import functools

import jax
import jax.numpy as jnp
from jax.experimental import pallas as pl
from jax.experimental.pallas import tpu as pltpu

EPS = 1e-05
INV_D = 30.0
H = W = 16
SKIP = 2
N_KEEP = (H - 2 * SKIP) * (W - 2 * SKIP)


def _moca_kernel(x_ref, emb_ref, codes_ref, bow_ref):
    xv = x_ref[0]
    n = jnp.sqrt(jnp.sum(xv * xv, axis=1, keepdims=True))
    xb = (xv * (INV_D / jnp.maximum(n, EPS))).astype(jnp.bfloat16)
    logits = jax.lax.dot_general(
        xb, emb_ref[...],
        dimension_numbers=(((1,), (1,)), ((), ())),
        preferred_element_type=jnp.float32,
    )
    e = jnp.exp(logits - INV_D)
    s = jnp.sum(e, axis=1, keepdims=True)
    r = 1.0 / s
    codes_ref[0] = e * r

    t = jax.lax.broadcasted_iota(jnp.int32, (1, 256), 1)
    tr = t // W
    tc = t % W
    keep = (tr >= SKIP) & (tr < H - SKIP) & (tc >= SKIP) & (tc < W - SKIP)
    w = jnp.where(keep, r.reshape(1, 256), 0.0) * (1.0 / N_KEEP)
    bow = jax.lax.dot_general(
        w, e,
        dimension_numbers=(((1,), (0,)), ((), ())),
        preferred_element_type=jnp.float32,
    )
    l1 = jnp.sum(jnp.abs(bow))
    bow_ref[0] = bow / jnp.maximum(l1, EPS)


@jax.jit
def kernel(x, embedding):
    B = x.shape[0]
    xs = x[:, 1:, :]
    L = xs.shape[1]
    K = embedding.shape[0]
    embedding = embedding.astype(jnp.bfloat16)
    codes, bow = pl.pallas_call(
        _moca_kernel,
        grid=(B,),
        in_specs=[
            pl.BlockSpec((1, L, xs.shape[2]), lambda b: (b, 0, 0)),
            pl.BlockSpec((K, xs.shape[2]), lambda b: (0, 0)),
        ],
        out_specs=[
            pl.BlockSpec((1, L, K), lambda b: (b, 0, 0)),
            pl.BlockSpec((1, 1, K), lambda b: (b, 0, 0)),
        ],
        out_shape=[
            jax.ShapeDtypeStruct((B, L, K), jnp.float32),
            jax.ShapeDtypeStruct((B, 1, K), jnp.float32),
        ],
    )(xs, embedding)
    return (bow.reshape(B, K), codes)
